# baseline (speedup 1.0000x reference)
import functools

import jax
import jax.numpy as jnp
from jax import lax
from jax.experimental import pallas as pl
from jax.experimental.pallas import tpu as pltpu
from jax.experimental.pallas import tpu_sc as plsc

N = 10000
K = 32
D = 128
C = 1000

NW = 32
BASE_CNT = N // NW
SCH = 8
NCHUNK = BASE_CNT // SCH
EXTRA_BASE = NW * BASE_CNT
LANES = 16
NV = D // LANES


def _sc_body(x1_hbm, sum_hbm, inb0, inb1, outb0, outb1,
             isem0, isem1, osem0, osem1):
    cid = lax.axis_index("c")
    sid = lax.axis_index("s")
    wid = sid * 2 + cid
    start = wid * BASE_CNT

    inbufs = (inb0, inb1)
    outbufs = (outb0, outb1)
    isems = (isem0, isem1)
    osems = (osem0, osem1)

    def in_copy(node0, slot):
        return pltpu.make_async_copy(
            x1_hbm.at[pl.ds(node0 * K, SCH * K)], inbufs[slot], isems[slot])

    def out_copy(node0, slot):
        return pltpu.make_async_copy(
            outbufs[slot], sum_hbm.at[pl.ds(node0, SCH)], osems[slot])

    def accumulate(inb, outb):
        for i in range(SCH):
            rb = i * K

            def kbody(k, accs):
                return tuple(accs[l] + inb[rb + k, pl.ds(l * LANES, LANES)]
                             for l in range(NV))

            accs = lax.fori_loop(
                0, K, kbody,
                tuple(jnp.zeros((LANES,), jnp.float32) for _ in range(NV)),
                unroll=8)
            for l in range(NV):
                outb[i, pl.ds(l * LANES, LANES)] = accs[l]

    in_copy(start, 0).start()
    in_copy(start + SCH, 1).start()

    def gbody(g, carry):
        for slot in (0, 1):
            chunk = 2 * g + slot
            node0 = start + chunk * SCH
            in_copy(node0, slot).wait()

            @pl.when(g > 0)
            def _():
                out_copy(node0 - 2 * SCH, slot).wait()

            accumulate(inbufs[slot], outbufs[slot])
            out_copy(node0, slot).start()

            @pl.when(chunk + 2 < NCHUNK)
            def _():
                in_copy(node0 + 2 * SCH, slot).start()
        return carry

    lax.fori_loop(0, (NCHUNK - 1) // 2, gbody, 0)

    last0 = start + (NCHUNK - 1) * SCH
    in_copy(last0, 0).wait()
    out_copy(last0 - 2 * SCH, 0).wait()
    accumulate(inb0, outb0)
    out_copy(last0, 0).start()

    out_copy(start + (NCHUNK - 2) * SCH, 1).wait()

    @pl.when(wid < 2)
    def _():
        node0 = EXTRA_BASE + wid * SCH
        in_copy(node0, 1).start()
        in_copy(node0, 1).wait()
        accumulate(inb1, outb1)
        out_copy(node0, 1).start()
        out_copy(node0, 1).wait()

    out_copy(last0, 0).wait()


@functools.partial(
    pl.kernel,
    out_type=jax.ShapeDtypeStruct((N, D), jnp.float32),
    mesh=plsc.VectorSubcoreMesh(core_axis_name="c", subcore_axis_name="s"),
    scratch_types=[
        pltpu.VMEM((SCH * K, D), jnp.float32),
        pltpu.VMEM((SCH * K, D), jnp.float32),
        pltpu.VMEM((SCH, D), jnp.float32),
        pltpu.VMEM((SCH, D), jnp.float32),
        pltpu.SemaphoreType.DMA,
        pltpu.SemaphoreType.DMA,
        pltpu.SemaphoreType.DMA,
        pltpu.SemaphoreType.DMA,
    ],
)
def _sc_segment_sum(x1_hbm, sum_hbm, inb0, inb1, outb0, outb1,
                    isem0, isem1, osem0, osem1):
    _sc_body(x1_hbm, sum_hbm, inb0, inb1, outb0, outb1,
             isem0, isem1, osem0, osem1)


TBLK = 1000


def _tc_body(x0_ref, s_ref, ws_ref, wn_ref, b_ref, fcw_ref, fcb_ref,
             out_ref, scores_ref):
    x0b = x0_ref[...]
    out = (
        jnp.dot(x0b, ws_ref[...], preferred_element_type=jnp.float32)
        + jnp.dot(s_ref[...], wn_ref[...], preferred_element_type=jnp.float32)
        + b_ref[...]
        + x0b
    )
    out_ref[...] = out
    scores_ref[...] = (
        jnp.dot(jnp.maximum(out, 0.0), fcw_ref[...],
                preferred_element_type=jnp.float32)
        + fcb_ref[...]
    )


def _tc_dense(x0, nsum, W_self, W_neigh_scaled, b2, fc_W, fcb2):
    return pl.pallas_call(
        _tc_body,
        grid=(N // TBLK,),
        in_specs=[
            pl.BlockSpec((TBLK, D), lambda i: (i, 0)),
            pl.BlockSpec((TBLK, D), lambda i: (i, 0)),
            pl.BlockSpec((D, D), lambda i: (0, 0)),
            pl.BlockSpec((D, D), lambda i: (0, 0)),
            pl.BlockSpec((1, D), lambda i: (0, 0)),
            pl.BlockSpec((D, C), lambda i: (0, 0)),
            pl.BlockSpec((1, C), lambda i: (0, 0)),
        ],
        out_specs=[
            pl.BlockSpec((TBLK, D), lambda i: (i, 0)),
            pl.BlockSpec((TBLK, C), lambda i: (i, 0)),
        ],
        out_shape=[
            jax.ShapeDtypeStruct((N, D), jnp.float32),
            jax.ShapeDtypeStruct((N, C), jnp.float32),
        ],
        compiler_params=pltpu.CompilerParams(
            dimension_semantics=("arbitrary",),
        ),
    )(x0, nsum, W_self, W_neigh_scaled, b2, fc_W, fcb2)


def kernel(x0, x1, W_self, W_neigh, b, fc_W, fc_b):
    nsum = _sc_segment_sum(x1)
    wn_scaled = W_neigh * (1.0 / K)
    b2 = b.reshape(1, D)
    fcb2 = fc_b.reshape(1, C)
    out, scores = _tc_dense(x0, nsum, W_self, wn_scaled, b2, fc_W, fcb2)
    return (out, scores)


# speedup vs baseline: 1.3218x; 1.3218x over previous
import functools

import jax
import jax.numpy as jnp
from jax import lax
from jax.experimental import pallas as pl
from jax.experimental.pallas import tpu as pltpu
from jax.experimental.pallas import tpu_sc as plsc

N = 10000
K = 32
D = 128
C = 1000

Q = 4096
LANES = 16
NV = D // LANES

NW = 32
WCNT = Q // NW
SCH = 4
NCHW = WCNT // SCH
SRING = 4


def _sc_body(x1_hbm, sum_hbm, inbufs, outbufs, isems, osems):
    cid = lax.axis_index("c")
    sid = lax.axis_index("s")
    wid = sid * 2 + cid
    start = wid * WCNT

    def in_copy(chunk, slot):
        node0 = start + chunk * SCH
        return pltpu.make_async_copy(
            x1_hbm.at[pl.ds(node0 * K, SCH * K)], inbufs[slot], isems[slot])

    def out_copy(chunk, slot):
        node0 = start + chunk * SCH
        return pltpu.make_async_copy(
            outbufs[slot], sum_hbm.at[pl.ds(node0, SCH)], osems[slot])

    def accumulate(inb, outb):
        for i in range(SCH):
            rb = i * K

            def kbody(k, accs):
                return tuple(accs[l] + inb[rb + k, pl.ds(l * LANES, LANES)]
                             for l in range(NV))

            accs = lax.fori_loop(
                0, K, kbody,
                tuple(jnp.zeros((LANES,), jnp.float32) for _ in range(NV)),
                unroll=8)
            for l in range(NV):
                outb[i, pl.ds(l * LANES, LANES)] = accs[l]

    for slot in range(SRING):
        in_copy(slot, slot).start()

    def gbody(g, carry):
        for slot in range(SRING):
            chunk = SRING * g + slot
            in_copy(chunk, slot).wait()

            @pl.when(g > 0)
            def _():
                out_copy(chunk - SRING, slot).wait()

            accumulate(inbufs[slot], outbufs[slot])
            out_copy(chunk, slot).start()

            @pl.when(chunk + SRING < NCHW)
            def _():
                in_copy(chunk + SRING, slot).start()
        return carry

    lax.fori_loop(0, NCHW // SRING, gbody, 0)

    for slot in range(SRING):
        out_copy(NCHW - SRING + slot, slot).wait()


@functools.partial(
    pl.kernel,
    out_type=jax.ShapeDtypeStruct((Q, D), jnp.float32),
    mesh=plsc.VectorSubcoreMesh(core_axis_name="c", subcore_axis_name="s"),
    scratch_types=[
        pltpu.VMEM((SCH * K, D), jnp.float32),
        pltpu.VMEM((SCH * K, D), jnp.float32),
        pltpu.VMEM((SCH * K, D), jnp.float32),
        pltpu.VMEM((SCH * K, D), jnp.float32),
        pltpu.VMEM((SCH, D), jnp.float32),
        pltpu.VMEM((SCH, D), jnp.float32),
        pltpu.VMEM((SCH, D), jnp.float32),
        pltpu.VMEM((SCH, D), jnp.float32),
        pltpu.SemaphoreType.DMA,
        pltpu.SemaphoreType.DMA,
        pltpu.SemaphoreType.DMA,
        pltpu.SemaphoreType.DMA,
        pltpu.SemaphoreType.DMA,
        pltpu.SemaphoreType.DMA,
        pltpu.SemaphoreType.DMA,
        pltpu.SemaphoreType.DMA,
    ],
)
def _sc_segment_sum(x1_hbm, sum_hbm,
                    ib0, ib1, ib2, ib3, ob0, ob1, ob2, ob3,
                    is0, is1, is2, is3, os0, os1, os2, os3):
    _sc_body(x1_hbm, sum_hbm, (ib0, ib1, ib2, ib3), (ob0, ob1, ob2, ob3),
             (is0, is1, is2, is3), (os0, os1, os2, os3))


CH1 = 984
NCH1 = (N - Q) // CH1
CHR1 = CH1 * K
TR = 2


def _tc1_body(ws_ref, wn_ref, b_ref, fcw_ref, fcb_ref,
              x0_hbm, x1_hbm, out_hbm, sc_hbm,
              x1buf, x0buf, outbuf, scbuf,
              in_sem, in0_sem, out_sem, sc_sem):

    def in_copies(c, slot):
        node0 = Q + c * CH1
        return (
            pltpu.make_async_copy(
                x1_hbm.at[pl.ds(node0 * K, CHR1)], x1buf.at[slot],
                in_sem.at[slot]),
            pltpu.make_async_copy(
                x0_hbm.at[pl.ds(node0, CH1)], x0buf.at[slot],
                in0_sem.at[slot]),
        )

    def out_copies(c, slot):
        node0 = Q + c * CH1
        return (
            pltpu.make_async_copy(
                outbuf.at[slot], out_hbm.at[pl.ds(node0, CH1)],
                out_sem.at[slot]),
            pltpu.make_async_copy(
                scbuf.at[slot], sc_hbm.at[pl.ds(node0, CH1)],
                sc_sem.at[slot]),
        )

    for r in range(TR):
        for cp in in_copies(r, r):
            cp.start()

    def step(c, carry):
        slot = lax.rem(c, TR)
        for cp in in_copies(c, slot):
            cp.wait()

        @pl.when(c >= TR)
        def _():
            for cp in out_copies(c - TR, slot):
                cp.wait()

        x0b = x0buf[slot]
        nsum = jnp.sum(x1buf[slot].reshape(CH1, K, D), axis=1)
        out = (
            jnp.dot(x0b, ws_ref[...], preferred_element_type=jnp.float32)
            + jnp.dot(nsum, wn_ref[...], preferred_element_type=jnp.float32)
            + b_ref[...]
            + x0b
        )
        outbuf[slot] = out
        scbuf[slot] = (
            jnp.dot(jnp.maximum(out, 0.0), fcw_ref[...],
                    preferred_element_type=jnp.float32)
            + fcb_ref[...]
        )
        for cp in out_copies(c, slot):
            cp.start()

        @pl.when(c + TR < NCH1)
        def _():
            for cp in in_copies(c + TR, slot):
                cp.start()

        return carry

    lax.fori_loop(0, NCH1, step, 0)

    for r in range(TR):
        c = NCH1 - TR + r
        for cp in out_copies(c, c % TR):
            cp.wait()


def _tc1(x0, x1, W_self, wn_scaled, b2, fc_W, fcb2):
    return pl.pallas_call(
        _tc1_body,
        in_specs=[
            pl.BlockSpec((D, D), lambda: (0, 0)),
            pl.BlockSpec((D, D), lambda: (0, 0)),
            pl.BlockSpec((1, D), lambda: (0, 0)),
            pl.BlockSpec((D, C), lambda: (0, 0)),
            pl.BlockSpec((1, C), lambda: (0, 0)),
            pl.BlockSpec(memory_space=pl.ANY),
            pl.BlockSpec(memory_space=pl.ANY),
        ],
        out_specs=[
            pl.BlockSpec(memory_space=pl.ANY),
            pl.BlockSpec(memory_space=pl.ANY),
        ],
        out_shape=[
            jax.ShapeDtypeStruct((N, D), jnp.float32),
            jax.ShapeDtypeStruct((N, C), jnp.float32),
        ],
        scratch_shapes=[
            pltpu.VMEM((TR, CHR1, D), jnp.float32),
            pltpu.VMEM((TR, CH1, D), jnp.float32),
            pltpu.VMEM((TR, CH1, D), jnp.float32),
            pltpu.VMEM((TR, CH1, C), jnp.float32),
            pltpu.SemaphoreType.DMA((TR,)),
            pltpu.SemaphoreType.DMA((TR,)),
            pltpu.SemaphoreType.DMA((TR,)),
            pltpu.SemaphoreType.DMA((TR,)),
        ],
    )(W_self, wn_scaled, b2, fc_W, fcb2, x0, x1)


TBLK2 = 512
NB2 = Q // TBLK2


def _tc2_body(x0_ref, s_ref, ws_ref, wn_ref, b_ref, fcw_ref, fcb_ref,
              outa_ref, sca_ref, out_ref, scores_ref):
    x0b = x0_ref[...]
    out = (
        jnp.dot(x0b, ws_ref[...], preferred_element_type=jnp.float32)
        + jnp.dot(s_ref[...], wn_ref[...], preferred_element_type=jnp.float32)
        + b_ref[...]
        + x0b
    )
    out_ref[...] = out
    scores_ref[...] = (
        jnp.dot(jnp.maximum(out, 0.0), fcw_ref[...],
                preferred_element_type=jnp.float32)
        + fcb_ref[...]
    )


def _tc2(x0, nsum, W_self, wn_scaled, b2, fc_W, fcb2, outa, scoresa):
    return pl.pallas_call(
        _tc2_body,
        grid=(NB2,),
        in_specs=[
            pl.BlockSpec((TBLK2, D), lambda i: (i, 0)),
            pl.BlockSpec((TBLK2, D), lambda i: (i, 0)),
            pl.BlockSpec((D, D), lambda i: (0, 0)),
            pl.BlockSpec((D, D), lambda i: (0, 0)),
            pl.BlockSpec((1, D), lambda i: (0, 0)),
            pl.BlockSpec((D, C), lambda i: (0, 0)),
            pl.BlockSpec((1, C), lambda i: (0, 0)),
            pl.BlockSpec(memory_space=pl.ANY),
            pl.BlockSpec(memory_space=pl.ANY),
        ],
        out_specs=[
            pl.BlockSpec((TBLK2, D), lambda i: (i, 0)),
            pl.BlockSpec((TBLK2, C), lambda i: (i, 0)),
        ],
        out_shape=[
            jax.ShapeDtypeStruct((N, D), jnp.float32),
            jax.ShapeDtypeStruct((N, C), jnp.float32),
        ],
        input_output_aliases={7: 0, 8: 1},
        compiler_params=pltpu.CompilerParams(
            dimension_semantics=("arbitrary",),
        ),
    )(x0, nsum, W_self, wn_scaled, b2, fc_W, fcb2, outa, scoresa)


def kernel(x0, x1, W_self, W_neigh, b, fc_W, fc_b):
    wn_scaled = W_neigh * (1.0 / K)
    b2 = b.reshape(1, D)
    fcb2 = fc_b.reshape(1, C)
    nsum = _sc_segment_sum(x1)
    outa, scoresa = _tc1(x0, x1, W_self, wn_scaled, b2, fc_W, fcb2)
    out, scores = _tc2(x0, nsum, W_self, wn_scaled, b2, fc_W, fcb2,
                       outa, scoresa)
    return (out, scores)


# speedup vs baseline: 1.3357x; 1.0105x over previous
import functools

import jax
import jax.numpy as jnp
from jax import lax
from jax.experimental import pallas as pl
from jax.experimental.pallas import tpu as pltpu
from jax.experimental.pallas import tpu_sc as plsc

N = 10000
K = 32
D = 128
C = 1000

Q = 3584
LANES = 16
NV = D // LANES

NW = 32
WCNT = Q // NW
SCH = 8
NCHW = WCNT // SCH
SRING = 2


def _sc_body(x1_hbm, sum_hbm, inbufs, outbufs, isems, osems):
    cid = lax.axis_index("c")
    sid = lax.axis_index("s")
    wid = sid * 2 + cid
    start = wid * WCNT

    def in_copy(chunk, slot):
        node0 = start + chunk * SCH
        return pltpu.make_async_copy(
            x1_hbm.at[pl.ds(node0 * K, SCH * K)], inbufs[slot], isems[slot])

    def out_copy(chunk, slot):
        node0 = start + chunk * SCH
        return pltpu.make_async_copy(
            outbufs[slot], sum_hbm.at[pl.ds(node0, SCH)], osems[slot])

    def accumulate(inb, outb):
        for i in range(SCH):
            rb = i * K

            def kbody(k, accs):
                return tuple(accs[l] + inb[rb + k, pl.ds(l * LANES, LANES)]
                             for l in range(NV))

            accs = lax.fori_loop(
                0, K, kbody,
                tuple(jnp.zeros((LANES,), jnp.float32) for _ in range(NV)),
                unroll=8)
            for l in range(NV):
                outb[i, pl.ds(l * LANES, LANES)] = accs[l]

    for slot in range(SRING):
        in_copy(slot, slot).start()

    def gbody(g, carry):
        for slot in range(SRING):
            chunk = SRING * g + slot
            in_copy(chunk, slot).wait()

            @pl.when(g > 0)
            def _():
                out_copy(chunk - SRING, slot).wait()

            accumulate(inbufs[slot], outbufs[slot])
            out_copy(chunk, slot).start()

            @pl.when(chunk + SRING < NCHW)
            def _():
                in_copy(chunk + SRING, slot).start()
        return carry

    lax.fori_loop(0, NCHW // SRING, gbody, 0)

    for slot in range(SRING):
        out_copy(NCHW - SRING + slot, slot).wait()


@functools.partial(
    pl.kernel,
    out_type=jax.ShapeDtypeStruct((Q, D), jnp.float32),
    mesh=plsc.VectorSubcoreMesh(core_axis_name="c", subcore_axis_name="s"),
    scratch_types=[
        pltpu.VMEM((SCH * K, D), jnp.float32),
        pltpu.VMEM((SCH * K, D), jnp.float32),
        pltpu.VMEM((SCH, D), jnp.float32),
        pltpu.VMEM((SCH, D), jnp.float32),
        pltpu.SemaphoreType.DMA,
        pltpu.SemaphoreType.DMA,
        pltpu.SemaphoreType.DMA,
        pltpu.SemaphoreType.DMA,
    ],
)
def _sc_segment_sum(x1_hbm, sum_hbm, ib0, ib1, ob0, ob1, is0, is1, os0, os1):
    _sc_body(x1_hbm, sum_hbm, (ib0, ib1), (ob0, ob1),
             (is0, is1), (os0, os1))


CH1 = 1072
_SIZES = [1072, 1072, 1072, 1072, 1072, 1056]
_BASES = [Q + sum(_SIZES[:i]) for i in range(len(_SIZES))]
TR = 2


def _tc1_body(ws_ref, wn_ref, b_ref, fcw_ref, fcb_ref,
              x0_hbm, x1_hbm, out_hbm, sc_hbm,
              x1buf, x0buf, outbuf, scbuf,
              in_sem, in0_sem, out_sem, sc_sem):

    def in_copies(c, slot):
        node0, sz = _BASES[c], _SIZES[c]
        return (
            pltpu.make_async_copy(
                x1_hbm.at[pl.ds(node0 * K, sz * K)],
                x1buf.at[slot, pl.ds(0, sz * K)], in_sem.at[slot]),
            pltpu.make_async_copy(
                x0_hbm.at[pl.ds(node0, sz)],
                x0buf.at[slot, pl.ds(0, sz)], in0_sem.at[slot]),
        )

    def out_copies(c, slot):
        node0, sz = _BASES[c], _SIZES[c]
        return (
            pltpu.make_async_copy(
                outbuf.at[slot, pl.ds(0, sz)],
                out_hbm.at[pl.ds(node0, sz)], out_sem.at[slot]),
            pltpu.make_async_copy(
                scbuf.at[slot, pl.ds(0, sz)],
                sc_hbm.at[pl.ds(node0, sz)], sc_sem.at[slot]),
        )

    for r in range(TR):
        for cp in in_copies(r, r):
            cp.start()

    for c in range(len(_SIZES)):
        slot = c % TR
        sz = _SIZES[c]
        for cp in in_copies(c, slot):
            cp.wait()
        if c >= TR:
            for cp in out_copies(c - TR, slot):
                cp.wait()

        x0b = x0buf[slot, 0:sz, :]
        nsum = jnp.sum(x1buf[slot, 0:sz * K, :].reshape(sz, K, D), axis=1)
        out = (
            jnp.dot(x0b, ws_ref[...], preferred_element_type=jnp.float32)
            + jnp.dot(nsum, wn_ref[...], preferred_element_type=jnp.float32)
            + b_ref[...]
            + x0b
        )
        outbuf[slot, 0:sz, :] = out
        scbuf[slot, 0:sz, :] = (
            jnp.dot(jnp.maximum(out, 0.0), fcw_ref[...],
                    preferred_element_type=jnp.float32)
            + fcb_ref[...]
        )
        for cp in out_copies(c, slot):
            cp.start()
        if c + TR < len(_SIZES):
            for cp in in_copies(c + TR, slot):
                cp.start()

    for c in (len(_SIZES) - 2, len(_SIZES) - 1):
        for cp in out_copies(c, c % TR):
            cp.wait()


def _tc1(x0, x1, W_self, wn_scaled, b2, fc_W, fcb2):
    return pl.pallas_call(
        _tc1_body,
        in_specs=[
            pl.BlockSpec((D, D), lambda: (0, 0)),
            pl.BlockSpec((D, D), lambda: (0, 0)),
            pl.BlockSpec((1, D), lambda: (0, 0)),
            pl.BlockSpec((D, C), lambda: (0, 0)),
            pl.BlockSpec((1, C), lambda: (0, 0)),
            pl.BlockSpec(memory_space=pl.ANY),
            pl.BlockSpec(memory_space=pl.ANY),
        ],
        out_specs=[
            pl.BlockSpec(memory_space=pl.ANY),
            pl.BlockSpec(memory_space=pl.ANY),
        ],
        out_shape=[
            jax.ShapeDtypeStruct((N, D), jnp.float32),
            jax.ShapeDtypeStruct((N, C), jnp.float32),
        ],
        scratch_shapes=[
            pltpu.VMEM((TR, CH1 * K, D), jnp.float32),
            pltpu.VMEM((TR, CH1, D), jnp.float32),
            pltpu.VMEM((TR, CH1, D), jnp.float32),
            pltpu.VMEM((TR, CH1, C), jnp.float32),
            pltpu.SemaphoreType.DMA((TR,)),
            pltpu.SemaphoreType.DMA((TR,)),
            pltpu.SemaphoreType.DMA((TR,)),
            pltpu.SemaphoreType.DMA((TR,)),
        ],
    )(W_self, wn_scaled, b2, fc_W, fcb2, x0, x1)


TBLK2 = 512
NB2 = Q // TBLK2


def _tc2_body(x0_ref, s_ref, ws_ref, wn_ref, b_ref, fcw_ref, fcb_ref,
              outa_ref, sca_ref, out_ref, scores_ref):
    x0b = x0_ref[...]
    out = (
        jnp.dot(x0b, ws_ref[...], preferred_element_type=jnp.float32)
        + jnp.dot(s_ref[...], wn_ref[...], preferred_element_type=jnp.float32)
        + b_ref[...]
        + x0b
    )
    out_ref[...] = out
    scores_ref[...] = (
        jnp.dot(jnp.maximum(out, 0.0), fcw_ref[...],
                preferred_element_type=jnp.float32)
        + fcb_ref[...]
    )


def _tc2(x0, nsum, W_self, wn_scaled, b2, fc_W, fcb2, outa, scoresa):
    return pl.pallas_call(
        _tc2_body,
        grid=(NB2,),
        in_specs=[
            pl.BlockSpec((TBLK2, D), lambda i: (i, 0)),
            pl.BlockSpec((TBLK2, D), lambda i: (i, 0)),
            pl.BlockSpec((D, D), lambda i: (0, 0)),
            pl.BlockSpec((D, D), lambda i: (0, 0)),
            pl.BlockSpec((1, D), lambda i: (0, 0)),
            pl.BlockSpec((D, C), lambda i: (0, 0)),
            pl.BlockSpec((1, C), lambda i: (0, 0)),
            pl.BlockSpec(memory_space=pl.ANY),
            pl.BlockSpec(memory_space=pl.ANY),
        ],
        out_specs=[
            pl.BlockSpec((TBLK2, D), lambda i: (i, 0)),
            pl.BlockSpec((TBLK2, C), lambda i: (i, 0)),
        ],
        out_shape=[
            jax.ShapeDtypeStruct((N, D), jnp.float32),
            jax.ShapeDtypeStruct((N, C), jnp.float32),
        ],
        input_output_aliases={7: 0, 8: 1},
        compiler_params=pltpu.CompilerParams(
            dimension_semantics=("arbitrary",),
        ),
    )(x0, nsum, W_self, wn_scaled, b2, fc_W, fcb2, outa, scoresa)


def kernel(x0, x1, W_self, W_neigh, b, fc_W, fc_b):
    wn_scaled = W_neigh * (1.0 / K)
    b2 = b.reshape(1, D)
    fcb2 = fc_b.reshape(1, C)
    nsum = _sc_segment_sum(x1)
    outa, scoresa = _tc1(x0, x1, W_self, wn_scaled, b2, fc_W, fcb2)
    out, scores = _tc2(x0, nsum, W_self, wn_scaled, b2, fc_W, fcb2,
                       outa, scoresa)
    return (out, scores)


# speedup vs baseline: 1.3583x; 1.0169x over previous
import functools

import jax
import jax.numpy as jnp
from jax import lax
from jax.experimental import pallas as pl
from jax.experimental.pallas import tpu as pltpu
from jax.experimental.pallas import tpu_sc as plsc

N = 10000
K = 32
D = 128
C = 1000

Q = 2048
LANES = 16
NV = D // LANES

NW = 32
WCNT = Q // NW
SCH = 8
NCHW = WCNT // SCH
SRING = 2


def _sc_body(x1_hbm, sum_hbm, inbufs, outbufs, isems, osems):
    cid = lax.axis_index("c")
    sid = lax.axis_index("s")
    wid = sid * 2 + cid
    start = wid * WCNT

    def in_copy(chunk, slot):
        node0 = start + chunk * SCH
        return pltpu.make_async_copy(
            x1_hbm.at[pl.ds(node0 * K, SCH * K)], inbufs[slot], isems[slot])

    def out_copy(chunk, slot):
        node0 = start + chunk * SCH
        return pltpu.make_async_copy(
            outbufs[slot], sum_hbm.at[pl.ds(node0, SCH)], osems[slot])

    def accumulate(inb, outb):
        for i in range(SCH):
            rb = i * K

            def kbody(k, accs):
                return tuple(accs[l] + inb[rb + k, pl.ds(l * LANES, LANES)]
                             for l in range(NV))

            accs = lax.fori_loop(
                0, K, kbody,
                tuple(jnp.zeros((LANES,), jnp.float32) for _ in range(NV)),
                unroll=8)
            for l in range(NV):
                outb[i, pl.ds(l * LANES, LANES)] = accs[l]

    for slot in range(SRING):
        in_copy(slot, slot).start()

    def gbody(g, carry):
        for slot in range(SRING):
            chunk = SRING * g + slot
            in_copy(chunk, slot).wait()

            @pl.when(g > 0)
            def _():
                out_copy(chunk - SRING, slot).wait()

            accumulate(inbufs[slot], outbufs[slot])
            out_copy(chunk, slot).start()

            @pl.when(chunk + SRING < NCHW)
            def _():
                in_copy(chunk + SRING, slot).start()
        return carry

    lax.fori_loop(0, NCHW // SRING, gbody, 0)

    for slot in range(SRING):
        out_copy(NCHW - SRING + slot, slot).wait()


@functools.partial(
    pl.kernel,
    out_type=jax.ShapeDtypeStruct((Q, D), jnp.float32),
    mesh=plsc.VectorSubcoreMesh(core_axis_name="c", subcore_axis_name="s"),
    scratch_types=[
        pltpu.VMEM((SCH * K, D), jnp.float32),
        pltpu.VMEM((SCH * K, D), jnp.float32),
        pltpu.VMEM((SCH, D), jnp.float32),
        pltpu.VMEM((SCH, D), jnp.float32),
        pltpu.SemaphoreType.DMA,
        pltpu.SemaphoreType.DMA,
        pltpu.SemaphoreType.DMA,
        pltpu.SemaphoreType.DMA,
    ],
)
def _sc_segment_sum(x1_hbm, sum_hbm, ib0, ib1, ob0, ob1, is0, is1, os0, os1):
    _sc_body(x1_hbm, sum_hbm, (ib0, ib1), (ob0, ob1),
             (is0, is1), (os0, os1))


CH1 = 1136
_SIZES = [1136] * 7
_BASES = [Q + sum(_SIZES[:i]) for i in range(len(_SIZES))]
TR = 2


def _tc1_body(ws_ref, wn_ref, b_ref, fcw_ref, fcb_ref,
              x0_hbm, x1_hbm, out_hbm, sc_hbm,
              x1buf, x0buf, outbuf, scbuf,
              in_sem, in0_sem, out_sem, sc_sem):

    def in_copies(c, slot):
        node0, sz = _BASES[c], _SIZES[c]
        return (
            pltpu.make_async_copy(
                x1_hbm.at[pl.ds(node0 * K, sz * K)],
                x1buf.at[slot, pl.ds(0, sz * K)], in_sem.at[slot]),
            pltpu.make_async_copy(
                x0_hbm.at[pl.ds(node0, sz)],
                x0buf.at[slot, pl.ds(0, sz)], in0_sem.at[slot]),
        )

    def out_copies(c, slot):
        node0, sz = _BASES[c], _SIZES[c]
        return (
            pltpu.make_async_copy(
                outbuf.at[slot, pl.ds(0, sz)],
                out_hbm.at[pl.ds(node0, sz)], out_sem.at[slot]),
            pltpu.make_async_copy(
                scbuf.at[slot, pl.ds(0, sz)],
                sc_hbm.at[pl.ds(node0, sz)], sc_sem.at[slot]),
        )

    for r in range(TR):
        for cp in in_copies(r, r):
            cp.start()

    for c in range(len(_SIZES)):
        slot = c % TR
        sz = _SIZES[c]
        for cp in in_copies(c, slot):
            cp.wait()
        if c >= TR:
            for cp in out_copies(c - TR, slot):
                cp.wait()

        x0b = x0buf[slot, 0:sz, :]
        nsum = jnp.sum(x1buf[slot, 0:sz * K, :].reshape(sz, K, D), axis=1)
        out = (
            jnp.dot(x0b, ws_ref[...], preferred_element_type=jnp.float32)
            + jnp.dot(nsum, wn_ref[...], preferred_element_type=jnp.float32)
            + b_ref[...]
            + x0b
        )
        outbuf[slot, 0:sz, :] = out
        scbuf[slot, 0:sz, :] = (
            jnp.dot(jnp.maximum(out, 0.0), fcw_ref[...],
                    preferred_element_type=jnp.float32)
            + fcb_ref[...]
        )
        for cp in out_copies(c, slot):
            cp.start()
        if c + TR < len(_SIZES):
            for cp in in_copies(c + TR, slot):
                cp.start()

    for c in (len(_SIZES) - 2, len(_SIZES) - 1):
        for cp in out_copies(c, c % TR):
            cp.wait()


def _tc1(x0, x1, W_self, wn_scaled, b2, fc_W, fcb2):
    return pl.pallas_call(
        _tc1_body,
        in_specs=[
            pl.BlockSpec((D, D), lambda: (0, 0)),
            pl.BlockSpec((D, D), lambda: (0, 0)),
            pl.BlockSpec((1, D), lambda: (0, 0)),
            pl.BlockSpec((D, C), lambda: (0, 0)),
            pl.BlockSpec((1, C), lambda: (0, 0)),
            pl.BlockSpec(memory_space=pl.ANY),
            pl.BlockSpec(memory_space=pl.ANY),
        ],
        out_specs=[
            pl.BlockSpec(memory_space=pl.ANY),
            pl.BlockSpec(memory_space=pl.ANY),
        ],
        out_shape=[
            jax.ShapeDtypeStruct((N, D), jnp.float32),
            jax.ShapeDtypeStruct((N, C), jnp.float32),
        ],
        scratch_shapes=[
            pltpu.VMEM((TR, CH1 * K, D), jnp.float32),
            pltpu.VMEM((TR, CH1, D), jnp.float32),
            pltpu.VMEM((TR, CH1, D), jnp.float32),
            pltpu.VMEM((TR, CH1, C), jnp.float32),
            pltpu.SemaphoreType.DMA((TR,)),
            pltpu.SemaphoreType.DMA((TR,)),
            pltpu.SemaphoreType.DMA((TR,)),
            pltpu.SemaphoreType.DMA((TR,)),
        ],
    )(W_self, wn_scaled, b2, fc_W, fcb2, x0, x1)


TBLK2 = 512
NB2 = Q // TBLK2


def _tc2_body(x0_ref, s_ref, ws_ref, wn_ref, b_ref, fcw_ref, fcb_ref,
              outa_ref, sca_ref, out_ref, scores_ref):
    x0b = x0_ref[...]
    out = (
        jnp.dot(x0b, ws_ref[...], preferred_element_type=jnp.float32)
        + jnp.dot(s_ref[...], wn_ref[...], preferred_element_type=jnp.float32)
        + b_ref[...]
        + x0b
    )
    out_ref[...] = out
    scores_ref[...] = (
        jnp.dot(jnp.maximum(out, 0.0), fcw_ref[...],
                preferred_element_type=jnp.float32)
        + fcb_ref[...]
    )


def _tc2(x0, nsum, W_self, wn_scaled, b2, fc_W, fcb2, outa, scoresa):
    return pl.pallas_call(
        _tc2_body,
        grid=(NB2,),
        in_specs=[
            pl.BlockSpec((TBLK2, D), lambda i: (i, 0)),
            pl.BlockSpec((TBLK2, D), lambda i: (i, 0)),
            pl.BlockSpec((D, D), lambda i: (0, 0)),
            pl.BlockSpec((D, D), lambda i: (0, 0)),
            pl.BlockSpec((1, D), lambda i: (0, 0)),
            pl.BlockSpec((D, C), lambda i: (0, 0)),
            pl.BlockSpec((1, C), lambda i: (0, 0)),
            pl.BlockSpec(memory_space=pl.ANY),
            pl.BlockSpec(memory_space=pl.ANY),
        ],
        out_specs=[
            pl.BlockSpec((TBLK2, D), lambda i: (i, 0)),
            pl.BlockSpec((TBLK2, C), lambda i: (i, 0)),
        ],
        out_shape=[
            jax.ShapeDtypeStruct((N, D), jnp.float32),
            jax.ShapeDtypeStruct((N, C), jnp.float32),
        ],
        input_output_aliases={7: 0, 8: 1},
        compiler_params=pltpu.CompilerParams(
            dimension_semantics=("arbitrary",),
        ),
    )(x0, nsum, W_self, wn_scaled, b2, fc_W, fcb2, outa, scoresa)


def kernel(x0, x1, W_self, W_neigh, b, fc_W, fc_b):
    wn_scaled = W_neigh * (1.0 / K)
    b2 = b.reshape(1, D)
    fcb2 = fc_b.reshape(1, C)
    nsum = _sc_segment_sum(x1)
    outa, scoresa = _tc1(x0, x1, W_self, wn_scaled, b2, fc_W, fcb2)
    out, scores = _tc2(x0, nsum, W_self, wn_scaled, b2, fc_W, fcb2,
                       outa, scoresa)
    return (out, scores)


# speedup vs baseline: 1.3693x; 1.0081x over previous
import functools

import jax
import jax.numpy as jnp
from jax import lax
from jax.experimental import pallas as pl
from jax.experimental.pallas import tpu as pltpu
from jax.experimental.pallas import tpu_sc as plsc

N = 10000
K = 32
D = 128
C = 1000

Q = 2048
LANES = 16
NV = D // LANES

NW = 32
WCNT = Q // NW
SCH = 8
NCHW = WCNT // SCH
SRING = 2


def _sc_body(x1_hbm, sum_hbm, inbufs, outbufs, isems, osems):
    cid = lax.axis_index("c")
    sid = lax.axis_index("s")
    wid = sid * 2 + cid
    start = wid * WCNT

    def in_copy(chunk, slot):
        node0 = start + chunk * SCH
        return pltpu.make_async_copy(
            x1_hbm.at[pl.ds(node0 * K, SCH * K)], inbufs[slot], isems[slot])

    def out_copy(chunk, slot):
        node0 = start + chunk * SCH
        return pltpu.make_async_copy(
            outbufs[slot], sum_hbm.at[pl.ds(node0, SCH)], osems[slot])

    def accumulate(inb, outb):
        for i in range(SCH):
            rb = i * K

            def kbody(k, accs):
                return tuple(accs[l] + inb[rb + k, pl.ds(l * LANES, LANES)]
                             for l in range(NV))

            accs = lax.fori_loop(
                0, K, kbody,
                tuple(jnp.zeros((LANES,), jnp.float32) for _ in range(NV)),
                unroll=8)
            for l in range(NV):
                outb[i, pl.ds(l * LANES, LANES)] = accs[l]

    for slot in range(SRING):
        in_copy(slot, slot).start()

    def gbody(g, carry):
        for slot in range(SRING):
            chunk = SRING * g + slot
            in_copy(chunk, slot).wait()

            @pl.when(g > 0)
            def _():
                out_copy(chunk - SRING, slot).wait()

            accumulate(inbufs[slot], outbufs[slot])
            out_copy(chunk, slot).start()

            @pl.when(chunk + SRING < NCHW)
            def _():
                in_copy(chunk + SRING, slot).start()
        return carry

    lax.fori_loop(0, NCHW // SRING, gbody, 0)

    for slot in range(SRING):
        out_copy(NCHW - SRING + slot, slot).wait()


@functools.partial(
    pl.kernel,
    out_type=jax.ShapeDtypeStruct((Q, D), jnp.float32),
    mesh=plsc.VectorSubcoreMesh(core_axis_name="c", subcore_axis_name="s"),
    scratch_types=[
        pltpu.VMEM((SCH * K, D), jnp.float32),
        pltpu.VMEM((SCH * K, D), jnp.float32),
        pltpu.VMEM((SCH, D), jnp.float32),
        pltpu.VMEM((SCH, D), jnp.float32),
        pltpu.SemaphoreType.DMA,
        pltpu.SemaphoreType.DMA,
        pltpu.SemaphoreType.DMA,
        pltpu.SemaphoreType.DMA,
    ],
)
def _sc_segment_sum(x1_hbm, sum_hbm, ib0, ib1, ob0, ob1, is0, is1, os0, os1):
    _sc_body(x1_hbm, sum_hbm, (ib0, ib1), (ob0, ob1),
             (is0, is1), (os0, os1))


CH1 = 1136
_SIZES = [1136] * 7
_BASES = [Q + sum(_SIZES[:i]) for i in range(len(_SIZES))]
TR = 2


def _tc1_body(ws_ref, wn_ref, b_ref, fcw_ref, fcb_ref,
              x0_hbm, x1_hbm, out_hbm, sc_hbm,
              x1buf, x0buf, outbuf, scbuf,
              in_sem, in0_sem, out_sem, sc_sem):

    def in_copies(c, slot):
        node0, sz = _BASES[c], _SIZES[c]
        return (
            pltpu.make_async_copy(
                x1_hbm.at[pl.ds(node0 * K, sz * K)],
                x1buf.at[slot, pl.ds(0, sz * K)], in_sem.at[slot]),
            pltpu.make_async_copy(
                x0_hbm.at[pl.ds(node0, sz)],
                x0buf.at[slot, pl.ds(0, sz)], in0_sem.at[slot]),
        )

    def out_copies(c, slot):
        node0, sz = _BASES[c], _SIZES[c]
        return (
            pltpu.make_async_copy(
                outbuf.at[slot, pl.ds(0, sz)],
                out_hbm.at[pl.ds(node0, sz)], out_sem.at[slot]),
            pltpu.make_async_copy(
                scbuf.at[slot, pl.ds(0, sz)],
                sc_hbm.at[pl.ds(node0, sz)], sc_sem.at[slot]),
        )

    for r in range(TR):
        for cp in in_copies(r, r):
            cp.start()

    for c in range(len(_SIZES)):
        slot = c % TR
        sz = _SIZES[c]
        for cp in in_copies(c, slot):
            cp.wait()
        if c >= TR:
            for cp in out_copies(c - TR, slot):
                cp.wait()

        x0b = x0buf[slot, 0:sz, :]
        nsum = jnp.sum(x1buf[slot, 0:sz * K, :].reshape(sz, K, D), axis=1)
        out = (
            jnp.dot(x0b, ws_ref[...], preferred_element_type=jnp.float32)
            + jnp.dot(nsum, wn_ref[...], preferred_element_type=jnp.float32)
            + b_ref[...]
            + x0b
        )
        outbuf[slot, 0:sz, :] = out
        scbuf[slot, 0:sz, :] = (
            jnp.dot(jnp.maximum(out, 0.0), fcw_ref[...],
                    preferred_element_type=jnp.float32)
            + fcb_ref[...]
        )
        for cp in out_copies(c, slot):
            cp.start()
        if c + TR < len(_SIZES):
            for cp in in_copies(c + TR, slot):
                cp.start()

    for c in (len(_SIZES) - 2, len(_SIZES) - 1):
        for cp in out_copies(c, c % TR):
            cp.wait()


def _tc1(x0, x1, W_self, wn_scaled, b2, fc_W, fcb2):
    return pl.pallas_call(
        _tc1_body,
        in_specs=[
            pl.BlockSpec((D, D), lambda: (0, 0)),
            pl.BlockSpec((D, D), lambda: (0, 0)),
            pl.BlockSpec((1, D), lambda: (0, 0)),
            pl.BlockSpec((D, C), lambda: (0, 0)),
            pl.BlockSpec((1, C), lambda: (0, 0)),
            pl.BlockSpec(memory_space=pl.ANY),
            pl.BlockSpec(memory_space=pl.ANY),
        ],
        out_specs=[
            pl.BlockSpec(memory_space=pl.ANY),
            pl.BlockSpec(memory_space=pl.ANY),
        ],
        out_shape=[
            jax.ShapeDtypeStruct((N, D), jnp.float32),
            jax.ShapeDtypeStruct((N, C), jnp.float32),
        ],
        scratch_shapes=[
            pltpu.VMEM((TR, CH1 * K, D), jnp.float32),
            pltpu.VMEM((TR, CH1, D), jnp.float32),
            pltpu.VMEM((TR, CH1, D), jnp.float32),
            pltpu.VMEM((TR, CH1, C), jnp.float32),
            pltpu.SemaphoreType.DMA((TR,)),
            pltpu.SemaphoreType.DMA((TR,)),
            pltpu.SemaphoreType.DMA((TR,)),
            pltpu.SemaphoreType.DMA((TR,)),
        ],
    )(W_self, wn_scaled, b2, fc_W, fcb2, x0, x1)


TBLK2 = 512
NB2 = Q // TBLK2


def _tc2_body(x0_ref, s_ref, ws_ref, wn_ref, b_ref, fcw_ref, fcb_ref,
              outa_ref, sca_ref, out_ref, scores_ref):
    x0b = x0_ref[...]
    out = (
        jnp.dot(x0b, ws_ref[...], preferred_element_type=jnp.float32)
        + jnp.dot(s_ref[...], wn_ref[...], preferred_element_type=jnp.float32)
        + b_ref[...]
        + x0b
    )
    out_ref[...] = out
    scores_ref[...] = (
        jnp.dot(jnp.maximum(out, 0.0), fcw_ref[...],
                preferred_element_type=jnp.float32)
        + fcb_ref[...]
    )


def _tc2(x0, nsum, W_self, wn_scaled, b2, fc_W, fcb2, outa, scoresa):
    return pl.pallas_call(
        _tc2_body,
        grid=(NB2,),
        in_specs=[
            pl.BlockSpec((TBLK2, D), lambda i: (i, 0)),
            pl.BlockSpec((TBLK2, D), lambda i: (i, 0)),
            pl.BlockSpec((D, D), lambda i: (0, 0)),
            pl.BlockSpec((D, D), lambda i: (0, 0)),
            pl.BlockSpec((1, D), lambda i: (0, 0)),
            pl.BlockSpec((D, C), lambda i: (0, 0)),
            pl.BlockSpec((1, C), lambda i: (0, 0)),
            pl.BlockSpec(memory_space=pl.ANY),
            pl.BlockSpec(memory_space=pl.ANY),
        ],
        out_specs=[
            pl.BlockSpec((TBLK2, D), lambda i: (i, 0)),
            pl.BlockSpec((TBLK2, C), lambda i: (i, 0)),
        ],
        out_shape=[
            jax.ShapeDtypeStruct((N, D), jnp.float32),
            jax.ShapeDtypeStruct((N, C), jnp.float32),
        ],
        input_output_aliases={7: 0, 8: 1},
        compiler_params=pltpu.CompilerParams(
            dimension_semantics=("arbitrary",),
        ),
    )(x0, nsum, W_self, wn_scaled, b2, fc_W, fcb2, outa, scoresa)


def kernel(x0, x1, W_self, W_neigh, b, fc_W, fc_b):
    wn_scaled = W_neigh * (1.0 / K)
    b2 = b.reshape(1, D)
    fcb2 = fc_b.reshape(1, C)
    outa, scoresa = _tc1(x0, x1, W_self, wn_scaled, b2, fc_W, fcb2)
    nsum = _sc_segment_sum(x1)
    out, scores = _tc2(x0, nsum, W_self, wn_scaled, b2, fc_W, fcb2,
                       outa, scoresa)
    return (out, scores)
